# baseline (speedup 1.0000x reference)
import jax
import jax.numpy as jnp
import numpy as np
from jax import lax
from jax.experimental import pallas as pl

BB = 32
EH = 64

_SPEC8K = [('wc1r', 64), ('wc2r', 48), ('wc4r', 48)]
_SPEC64 = [('we0a_s', 64), ('we0a_d', 64), ('we0b', 64),
           ('we1a_s', 48), ('we1a_d', 48), ('we1b', 64),
           ('we2a_s', 48), ('we2a_d', 48), ('we2b', 64),
           ('we4a_s', 16), ('we4a_d', 16), ('we4b', 64),
           ('wfe0a', 64), ('wfe0b', 64), ('wfe0p', 128),
           ('wl_s', 8), ('wl_d', 8), ('bfe0', 8)]
_SPEC96 = [('wu1', 128), ('bu1', 8), ('wu2', 128), ('bu2', 8)]
_SPEC128 = [('bc1m', 64), ('rc1', 64), ('cb1', 8),
            ('bc2m', 48), ('rc2', 48), ('cb2', 8),
            ('bc4m', 48), ('rc4', 48), ('cb4', 8),
            ('wf0', 128), ('bf0', 8)]
_SPEC384 = [('wi1', 128), ('bi1', 8)]
_SPEC192 = [('wi2', 384), ('bi2', 8)]
_SPEC16 = [('wf1', 128), ('bf1', 8), ('wfe1', 64), ('bfe1', 8)]


def _offsets(spec):
    out, o = {}, 0
    for name, rows in spec:
        out[name] = (o, rows)
        o += rows
    return out, o


_OFF8K, _N8K = _offsets(_SPEC8K)
_OFF64, _N64 = _offsets(_SPEC64)
_OFF96, _N96 = _offsets(_SPEC96)
_OFF128, _N128 = _offsets(_SPEC128)
_OFF384, _N384 = _offsets(_SPEC384)
_OFF192, _N192 = _offsets(_SPEC192)
_OFF16, _N16 = _offsets(_SPEC16)


def _leaky(x):
    return jnp.maximum(x, 0.05 * x)


def _pair_mask(n, rows):
    r = lax.broadcasted_iota(jnp.int32, (rows, 1), 0)
    p = r % (n * n)
    return jnp.where(p // n != p % n, 1.0, 0.0).astype(jnp.float32)


def _pair_add(u, v, n, d):
    bb = u.shape[0] // n
    u4 = u.reshape(bb, n, 1, d)
    v4 = v.reshape(bb, 1, n, d)
    return jnp.broadcast_to(u4, (bb, n, n, d)) + jnp.broadcast_to(v4, (bb, n, n, d))


def _edge_attr(x2, n, Wa_s, Wa_d, Wb):
    u = x2 @ Wa_s
    v = x2 @ Wa_d
    pre = _leaky(_pair_add(u, v, n, EH)).reshape(-1, EH)
    return _leaky(pre @ Wb)


def _conv(xf, n, din, e_full, Wcr, bcm, R, cb):
    bb = xf.shape[0]
    x2 = xf.reshape(bb * n, din)
    xb = x2.astype(jnp.bfloat16)
    Y = jnp.dot(xb, Wcr, preferred_element_type=jnp.float32)
    Y = Y.reshape(bb * n, EH, 128)
    E3 = e_full.reshape(bb * n, n, EH)
    Z = lax.dot_general(E3, Y, (((2,), (1,)), ((0,), (0,))))
    agg = jnp.sum(Z.reshape(bb, n, n, 128), axis=1).reshape(bb * n, 128)
    xex = (jnp.sum(xf, axis=1, keepdims=True) - xf).reshape(bb * n, din)
    agg = (agg + xex @ bcm) * (1.0 / (n - 1))
    return _leaky(x2 @ R + cb + agg).reshape(bb, n, 128)


def _body(z_ref, p8k_ref, p64_ref, p96_ref, p128_ref, p384_ref, p192_ref,
          p16_ref, blink_ref, node_ref, edge_ref):
    def g(ref, off, name, rows=None):
        o, r = off[name]
        return ref[o:o + (rows if rows is not None else r), :]

    z = z_ref[...]
    h = _leaky(z @ g(p384_ref, _OFF384, 'wi1') + g(p384_ref, _OFF384, 'bi1', 1))
    x0 = _leaky((h @ g(p192_ref, _OFF192, 'wi2')
                 + g(p192_ref, _OFF192, 'bi2', 1)).reshape(BB, 3, 64))

    x0_2 = x0.reshape(BB * 3, 64)
    e0 = _edge_attr(x0_2, 3, g(p64_ref, _OFF64, 'we0a_s'),
                    g(p64_ref, _OFF64, 'we0a_d'), g(p64_ref, _OFF64, 'we0b'))
    ga = jnp.sum(x0_2 * g(p64_ref, _OFF64, 'wl_s', 1), axis=-1, keepdims=True)
    gb = jnp.sum(x0_2 * g(p64_ref, _OFF64, 'wl_d', 1), axis=-1, keepdims=True)
    gate = jax.nn.sigmoid(
        _pair_add(ga, gb, 3, 1).reshape(BB * 9, 1) + blink_ref[...])
    e0 = e0 * (gate * _pair_mask(3, BB * 9))
    x1 = _conv(x0, 3, 64, e0, g(p8k_ref, _OFF8K, 'wc1r'),
               g(p128_ref, _OFF128, 'bc1m'), g(p128_ref, _OFF128, 'rc1'),
               g(p128_ref, _OFF128, 'cb1', 1))

    x1u = (x1.reshape(BB * 3, 128) @ g(p96_ref, _OFF96, 'wu1')
           + g(p96_ref, _OFF96, 'bu1', 1))
    x1u = _leaky(x1u.reshape(BB * 3, 2, 48)).reshape(BB, 6, 48)
    e1 = _edge_attr(x1u.reshape(BB * 6, 48), 6, g(p64_ref, _OFF64, 'we1a_s'),
                    g(p64_ref, _OFF64, 'we1a_d'), g(p64_ref, _OFF64, 'we1b'))
    e1 = e1 * _pair_mask(6, BB * 36)
    x2 = _conv(x1u, 6, 48, e1, g(p8k_ref, _OFF8K, 'wc2r'),
               g(p128_ref, _OFF128, 'bc2m'), g(p128_ref, _OFF128, 'rc2'),
               g(p128_ref, _OFF128, 'cb2', 1))

    x2u = (x2.reshape(BB * 6, 128) @ g(p96_ref, _OFF96, 'wu2')
           + g(p96_ref, _OFF96, 'bu2', 1))
    x2u = _leaky(x2u.reshape(BB * 6, 2, 48)).reshape(BB, 12, 48)
    e2 = _edge_attr(x2u.reshape(BB * 12, 48), 12, g(p64_ref, _OFF64, 'we2a_s'),
                    g(p64_ref, _OFF64, 'we2a_d'), g(p64_ref, _OFF64, 'we2b'))
    e2 = e2 * _pair_mask(12, BB * 144)
    x3 = _conv(x2u, 12, 48, e2, g(p8k_ref, _OFF8K, 'wc4r'),
               g(p128_ref, _OFF128, 'bc4m'), g(p128_ref, _OFF128, 'rc4'),
               g(p128_ref, _OFF128, 'cb4', 1))

    h0 = _leaky(x3.reshape(BB * 12, 128) @ g(p128_ref, _OFF128, 'wf0')
                + g(p128_ref, _OFF128, 'bf0', 1))
    node = h0 @ g(p16_ref, _OFF16, 'wf1') + g(p16_ref, _OFF16, 'bf1', 1)
    node_ref[...] = node.reshape(BB, 12, 16)

    e4 = _edge_attr(node, 12, g(p64_ref, _OFF64, 'we4a_s'),
                    g(p64_ref, _OFF64, 'we4a_d'), g(p64_ref, _OFF64, 'we4b'))
    hp = h0 @ g(p64_ref, _OFF64, 'wfe0p')
    ef = _leaky(e2 @ g(p64_ref, _OFF64, 'wfe0a')
                + e4 @ g(p64_ref, _OFF64, 'wfe0b')
                + _pair_add(hp, hp, 12, EH).reshape(BB * 144, EH)
                + g(p64_ref, _OFF64, 'bfe0', 1))
    edge = ef @ g(p16_ref, _OFF16, 'wfe1') + g(p16_ref, _OFF16, 'bfe1', 1)
    edge_ref[...] = edge.reshape(BB, 144, 16)


def _full(shape):
    nd = len(shape)
    return pl.BlockSpec(shape, lambda i: (0,) * nd)


def kernel(z, Wi1, bi1, Wi2, bi2, Wlink, blink, We0a, We0b, Wc1, bc1, Rc1, cb1,
           Wu1, bu1, We1a, We1b, Wc2, bc2, Rc2, cb2, Wu2, bu2, We2a, We2b,
           Wc4, bc4, Rc4, cb4, Wf0, bf0, Wf1, bf1, We4a, We4b, Wfe0, bfe0,
           Wfe1, bfe1):
    B = z.shape[0]

    def relayout(Wc, din, dout):
        w = Wc.reshape(EH, din, dout).transpose(1, 0, 2)
        w = jnp.pad(w, ((0, 0), (0, 0), (0, 128 - dout)))
        return w.reshape(din, EH * 128)

    def padcols(w, width):
        return jnp.pad(w, ((0, 0), (0, width - w.shape[1])))

    def padrows(w, height):
        return jnp.pad(w, ((0, height - w.shape[0]), (0, 0)))

    row = lambda v: v.reshape(1, -1)

    def pack(spec, pieces, dtype, width):
        arrs = []
        for name, rows in spec:
            a = pieces[name].astype(dtype)
            if a.shape[1] < width:
                a = padcols(a, width)
            if a.shape[0] < rows:
                a = padrows(a, rows)
            assert a.shape == (rows, width), (name, a.shape)
            arrs.append(a)
        return jnp.concatenate(arrs, axis=0)

    p8k = pack(_SPEC8K, {
        'wc1r': relayout(Wc1, 64, 64),
        'wc2r': relayout(Wc2, 48, 64),
        'wc4r': relayout(Wc4, 48, 64),
    }, jnp.bfloat16, EH * 128)
    p64 = pack(_SPEC64, {
        'we0a_s': We0a[:64], 'we0a_d': We0a[64:], 'we0b': We0b,
        'we1a_s': We1a[:48], 'we1a_d': We1a[48:], 'we1b': We1b,
        'we2a_s': We2a[:48], 'we2a_d': We2a[48:], 'we2b': We2b,
        'we4a_s': We4a[:16], 'we4a_d': We4a[16:], 'we4b': We4b,
        'wfe0a': Wfe0[:EH], 'wfe0b': Wfe0[EH:2 * EH], 'wfe0p': 0.5 * Wfe0[2 * EH:],
        'wl_s': row(Wlink[:64, 0]), 'wl_d': row(Wlink[64:, 0]),
        'bfe0': row(bfe0),
    }, jnp.float32, 64)
    p96 = pack(_SPEC96, {
        'wu1': padrows(Wu1, 128), 'bu1': row(bu1),
        'wu2': padrows(Wu2, 128), 'bu2': row(bu2),
    }, jnp.float32, 96)
    p128 = pack(_SPEC128, {
        'bc1m': bc1.reshape(64, 64), 'rc1': Rc1, 'cb1': row(cb1),
        'bc2m': bc2.reshape(48, 64), 'rc2': Rc2, 'cb2': row(cb2),
        'bc4m': bc4.reshape(48, 64), 'rc4': Rc4, 'cb4': row(cb4),
        'wf0': padrows(Wf0, 128), 'bf0': row(bf0),
    }, jnp.float32, 128)
    p384 = pack(_SPEC384, {'wi1': Wi1, 'bi1': row(bi1)}, jnp.float32, 384)
    p192 = pack(_SPEC192, {'wi2': Wi2, 'bi2': row(bi2)}, jnp.float32, 192)
    p16 = pack(_SPEC16, {
        'wf1': Wf1, 'bf1': row(bf1),
        'wfe1': Wfe1, 'bfe1': row(bfe1),
    }, jnp.float32, 16)

    ins = [z, p8k, p64, p96, p128, p384, p192, p16, row(blink)]
    in_specs = [pl.BlockSpec((BB, 128), lambda i: (i, 0))]
    in_specs += [_full(a.shape) for a in ins[1:]]

    node_out, edge_full = pl.pallas_call(
        _body,
        grid=(B // BB,),
        in_specs=in_specs,
        out_specs=[
            pl.BlockSpec((BB, 12, 16), lambda i: (i, 0, 0)),
            pl.BlockSpec((BB, 144, 16), lambda i: (i, 0, 0)),
        ],
        out_shape=[
            jax.ShapeDtypeStruct((B, 12, 16), jnp.float32),
            jax.ShapeDtypeStruct((B, 144, 16), jnp.float32),
        ],
    )(*ins)

    offdiag = np.array([i * 12 + j for i in range(12) for j in range(12)
                        if i != j], dtype=np.int32)
    return node_out, edge_full[:, offdiag, :4]


# speedup vs baseline: 1.0454x; 1.0454x over previous
import functools

import jax
import jax.numpy as jnp
import numpy as np
from jax import lax
from jax.experimental import pallas as pl
from jax.experimental.pallas import tpu as pltpu

BB = 16
EH = 64


def _leaky(x):
    return jnp.maximum(x, 0.05 * x)


def _pair_mask(n, rows):
    r = lax.broadcasted_iota(jnp.int32, (rows, 1), 0)
    p = r % (n * n)
    return jnp.where(p // n != p % n, 1.0, 0.0).astype(jnp.float32)


def _pair_add(u, v, n, d):
    bb = u.shape[0] // n
    u4 = u.reshape(bb, n, 1, d)
    v4 = v.reshape(bb, 1, n, d)
    return jnp.broadcast_to(u4, (bb, n, n, d)) + jnp.broadcast_to(v4, (bb, n, n, d))


def _edge_attr(x2, n, Wa_s, Wa_d, Wb):
    u = x2 @ Wa_s
    v = x2 @ Wa_d
    pre = _leaky(_pair_add(u, v, n, EH)).reshape(-1, EH)
    return _leaky(pre @ Wb)


def _conv(xf, n, din, e_full, Wcr, bcm, R, cb):
    bb = xf.shape[0]
    x2 = xf.reshape(bb * n, din)
    Y = (x2 @ Wcr).reshape(bb * n, EH, 64)
    E3 = e_full.reshape(bb * n, n, EH)
    Z = lax.dot_general(E3, Y, (((2,), (1,)), ((0,), (0,))))
    agg = jnp.sum(Z.reshape(bb, n, n, 64), axis=1).reshape(bb * n, 64)
    xex = (jnp.sum(xf, axis=1, keepdims=True) - xf).reshape(bb * n, din)
    agg = (agg + xex @ bcm) * (1.0 / (n - 1))
    return _leaky(x2 @ R + cb + agg).reshape(bb, n, 64)


def _body(z_ref, wi1_ref, bi1_ref, wi2_ref, bi2_ref, wl_s_ref, wl_d_ref,
          blink_ref,
          we0a_s_ref, we0a_d_ref, we0b_ref, wc1r_ref, bc1m_ref, rc1_ref,
          cb1_ref, wu1_ref, bu1_ref, we1a_s_ref, we1a_d_ref, we1b_ref,
          wc2r_ref, bc2m_ref, rc2_ref, cb2_ref, wu2_ref, bu2_ref,
          we2a_s_ref, we2a_d_ref, we2b_ref, wc4r_ref, bc4m_ref, rc4_ref,
          cb4_ref, wf0_ref, bf0_ref, wf1_ref, bf1_ref,
          we4a_s_ref, we4a_d_ref, we4b_ref,
          wfe0a_ref, wfe0b_ref, wfe0p_ref, bfe0_ref,
          wfe1_ref, bfe1_ref, node_ref, edge_ref):
    z = z_ref[...]
    h = _leaky(z @ wi1_ref[...] + bi1_ref[...])
    x0 = _leaky((h @ wi2_ref[...] + bi2_ref[...]).reshape(BB, 3, 64))

    x0_2 = x0.reshape(BB * 3, 64)
    e0 = _edge_attr(x0_2, 3, we0a_s_ref[...], we0a_d_ref[...], we0b_ref[...])
    ga = jnp.sum(x0_2 * wl_s_ref[...], axis=-1, keepdims=True)
    gb = jnp.sum(x0_2 * wl_d_ref[...], axis=-1, keepdims=True)
    gate = jax.nn.sigmoid(
        _pair_add(ga, gb, 3, 1).reshape(BB * 9, 1) + blink_ref[...])
    e0 = e0 * (gate * _pair_mask(3, BB * 9))
    x1 = _conv(x0, 3, 64, e0, wc1r_ref[...], bc1m_ref[...],
               rc1_ref[...], cb1_ref[...])

    x1u = (x1.reshape(BB * 3, 64) @ wu1_ref[...] + bu1_ref[...])
    x1u = _leaky(x1u.reshape(BB * 3, 2, 48)).reshape(BB, 6, 48)
    e1 = _edge_attr(x1u.reshape(BB * 6, 48), 6, we1a_s_ref[...],
                    we1a_d_ref[...], we1b_ref[...])
    e1 = e1 * _pair_mask(6, BB * 36)
    x2 = _conv(x1u, 6, 48, e1, wc2r_ref[...], bc2m_ref[...],
               rc2_ref[...], cb2_ref[...])

    x2u = (x2.reshape(BB * 6, 64) @ wu2_ref[...] + bu2_ref[...])
    x2u = _leaky(x2u.reshape(BB * 6, 2, 48)).reshape(BB, 12, 48)
    e2 = _edge_attr(x2u.reshape(BB * 12, 48), 12, we2a_s_ref[...],
                    we2a_d_ref[...], we2b_ref[...])
    e2 = e2 * _pair_mask(12, BB * 144)
    x3 = _conv(x2u, 12, 48, e2, wc4r_ref[...], bc4m_ref[...],
               rc4_ref[...], cb4_ref[...])

    h0 = _leaky(x3.reshape(BB * 12, 64) @ wf0_ref[...] + bf0_ref[...])
    node = h0 @ wf1_ref[...] + bf1_ref[...]
    node_ref[...] = node.reshape(BB, 12, 16)

    e4 = _edge_attr(node, 12, we4a_s_ref[...], we4a_d_ref[...],
                    we4b_ref[...])
    hp = h0 @ wfe0p_ref[...]
    ef = _leaky(e2 @ wfe0a_ref[...] + e4 @ wfe0b_ref[...]
                + _pair_add(hp, hp, 12, EH).reshape(BB * 144, EH)
                + bfe0_ref[...])
    edge = ef @ wfe1_ref[...] + bfe1_ref[...]
    edge_ref[...] = edge.reshape(BB, 144, 4)


def _full(shape):
    nd = len(shape)
    return pl.BlockSpec(shape, lambda i: (0,) * nd)


def kernel(z, Wi1, bi1, Wi2, bi2, Wlink, blink, We0a, We0b, Wc1, bc1, Rc1, cb1,
           Wu1, bu1, We1a, We1b, Wc2, bc2, Rc2, cb2, Wu2, bu2, We2a, We2b,
           Wc4, bc4, Rc4, cb4, Wf0, bf0, Wf1, bf1, We4a, We4b, Wfe0, bfe0,
           Wfe1, bfe1):
    B = z.shape[0]

    def relayout(Wc, din, dout):
        w = Wc.reshape(EH, din, dout).transpose(1, 0, 2)
        return w.reshape(din, EH * dout)

    padcols = lambda w: w
    padrows = lambda w: w

    wc1r = relayout(Wc1, 64, 64)
    wc2r = relayout(Wc2, 48, 64)
    wc4r = relayout(Wc4, 48, 64)
    row = lambda v: v.reshape(1, -1)
    half = lambda w: (w[: w.shape[0] // 2], w[w.shape[0] // 2:])
    we0a_s, we0a_d = half(We0a)
    we1a_s, we1a_d = half(We1a)
    we2a_s, we2a_d = half(We2a)
    we4a_s, we4a_d = half(We4a)
    wl_s, wl_d = row(Wlink[:64, 0]), row(Wlink[64:, 0])
    wfe0a, wfe0b, wfe0p = Wfe0[:EH], Wfe0[EH:2 * EH], 0.5 * Wfe0[2 * EH:]
    ins = [z, Wi1, row(bi1), Wi2, row(bi2), wl_s, wl_d, row(blink),
           we0a_s, we0a_d, We0b, wc1r, padcols(bc1.reshape(64, 64)),
           padcols(Rc1), padcols(row(cb1)),
           padrows(Wu1), row(bu1), we1a_s, we1a_d, We1b, wc2r,
           padcols(bc2.reshape(48, 64)), padcols(Rc2), padcols(row(cb2)),
           padrows(Wu2), row(bu2), we2a_s, we2a_d, We2b,
           wc4r, padcols(bc4.reshape(48, 64)), padcols(Rc4), padcols(row(cb4)),
           padrows(Wf0), row(bf0),
           Wf1, row(bf1), we4a_s, we4a_d, We4b,
           wfe0a, wfe0b, wfe0p, row(bfe0), Wfe1, row(bfe1)]

    in_specs = [pl.BlockSpec((BB, 128), lambda i: (i, 0))]
    in_specs += [_full(a.shape) for a in ins[1:]]

    node_out, edge_full = pl.pallas_call(
        _body,
        grid=(B // BB,),
        in_specs=in_specs,
        out_specs=[
            pl.BlockSpec((BB, 12, 16), lambda i: (i, 0, 0)),
            pl.BlockSpec((BB, 144, 4), lambda i: (i, 0, 0)),
        ],
        out_shape=[
            jax.ShapeDtypeStruct((B, 12, 16), jnp.float32),
            jax.ShapeDtypeStruct((B, 144, 4), jnp.float32),
        ],
    )(*ins)

    offdiag = np.array([i * 12 + j for i in range(12) for j in range(12)
                        if i != j], dtype=np.int32)
    return node_out, edge_full[:, offdiag, :]


# speedup vs baseline: 1.1411x; 1.0915x over previous
import functools

import jax
import jax.numpy as jnp
import numpy as np
from jax import lax
from jax.experimental import pallas as pl
from jax.experimental.pallas import tpu as pltpu

BB = 16
EH = 64


def _leaky(x):
    return jnp.where(x >= 0, x, 0.05 * x)


def _pair_mask(n, rows):
    r = lax.broadcasted_iota(jnp.int32, (rows, 1), 0)
    p = r % (n * n)
    return jnp.where(p // n != p % n, 1.0, 0.0).astype(jnp.float32)


def _pairs(xf, n, d):
    bb = xf.shape[0]
    xs = jnp.broadcast_to(xf[:, :, None, :], (bb, n, n, d)).reshape(bb * n * n, d)
    xd = jnp.broadcast_to(xf[:, None, :, :], (bb, n, n, d)).reshape(bb * n * n, d)
    return xs, xd


def _edge_attr(xf, n, d, Wa, Wb):
    xs, xd = _pairs(xf, n, d)
    cat = jnp.concatenate([xs, xd], axis=-1)
    return cat, _leaky(_leaky(cat @ Wa) @ Wb)


def _conv(xf, n, din, dout, e_full, Wcr, bcm, R, cb):
    bb = xf.shape[0]
    x2 = xf.reshape(bb * n, din)
    Y = (x2 @ Wcr).reshape(bb * n, EH, dout)
    E3 = e_full.reshape(bb * n, n, EH)
    Z = lax.dot_general(E3, Y, (((2,), (1,)), ((0,), (0,))))
    agg = jnp.sum(Z.reshape(bb, n, n, dout), axis=1).reshape(bb * n, dout)
    xex = (jnp.sum(xf, axis=1, keepdims=True) - xf).reshape(bb * n, din)
    agg = (agg + xex @ bcm) * (1.0 / (n - 1))
    return _leaky(x2 @ R + cb + agg).reshape(bb, n, dout)


def _body(z_ref, wi1_ref, bi1_ref, wi2_ref, bi2_ref, wl_ref, blink_ref,
          we0a_ref, we0b_ref, wc1r_ref, bc1m_ref, rc1_ref, cb1_ref,
          wu1_ref, bu1_ref, we1a_ref, we1b_ref, wc2r_ref, bc2m_ref,
          rc2_ref, cb2_ref, wu2_ref, bu2_ref, we2a_ref, we2b_ref,
          wc4r_ref, bc4m_ref, rc4_ref, cb4_ref, wf0_ref, bf0_ref,
          wf1_ref, bf1_ref, we4a_ref, we4b_ref, wfe0_ref, bfe0_ref,
          wfe1_ref, bfe1_ref, node_ref, edge_ref):
    z = z_ref[...]
    h = _leaky(z @ wi1_ref[...] + bi1_ref[...])
    x0 = _leaky((h @ wi2_ref[...] + bi2_ref[...]).reshape(BB, 3, 64))

    cat0, e0 = _edge_attr(x0, 3, 64, we0a_ref[...], we0b_ref[...])
    gate = jax.nn.sigmoid(
        jnp.sum(cat0 * wl_ref[...], axis=-1, keepdims=True) + blink_ref[...])
    e0 = e0 * (gate * _pair_mask(3, BB * 9))
    x1 = _conv(x0, 3, 64, 64, e0, wc1r_ref[...], bc1m_ref[...],
               rc1_ref[...], cb1_ref[...])

    x1u = (x1.reshape(BB * 3, 64) @ wu1_ref[...] + bu1_ref[...])
    x1u = _leaky(x1u.reshape(BB * 3, 2, 48)).reshape(BB, 6, 48)
    _, e1 = _edge_attr(x1u, 6, 48, we1a_ref[...], we1b_ref[...])
    e1 = e1 * _pair_mask(6, BB * 36)
    x2 = _conv(x1u, 6, 48, 64, e1, wc2r_ref[...], bc2m_ref[...],
               rc2_ref[...], cb2_ref[...])

    x2u = (x2.reshape(BB * 6, 64) @ wu2_ref[...] + bu2_ref[...])
    x2u = _leaky(x2u.reshape(BB * 6, 2, 48)).reshape(BB, 12, 48)
    _, e2 = _edge_attr(x2u, 12, 48, we2a_ref[...], we2b_ref[...])
    e2 = e2 * _pair_mask(12, BB * 144)
    x3 = _conv(x2u, 12, 48, 64, e2, wc4r_ref[...], bc4m_ref[...],
               rc4_ref[...], cb4_ref[...])

    h0 = _leaky(x3.reshape(BB * 12, 64) @ wf0_ref[...] + bf0_ref[...])
    node = h0 @ wf1_ref[...] + bf1_ref[...]
    node_ref[...] = node.reshape(BB, 12, 16)

    _, e4 = _edge_attr(node.reshape(BB, 12, 16), 12, 16,
                       we4a_ref[...], we4b_ref[...])
    hs, hd = _pairs(h0.reshape(BB, 12, 64), 12, 64)
    pairf = 0.5 * (hs + hd)
    ef = _leaky(jnp.concatenate([e2, e4, pairf], axis=-1) @ wfe0_ref[...]
                + bfe0_ref[...])
    edge = ef @ wfe1_ref[...] + bfe1_ref[...]
    edge_ref[...] = edge.reshape(BB, 144, 4)


def _full(shape):
    nd = len(shape)
    return pl.BlockSpec(shape, lambda i: (0,) * nd)


def kernel(z, Wi1, bi1, Wi2, bi2, Wlink, blink, We0a, We0b, Wc1, bc1, Rc1, cb1,
           Wu1, bu1, We1a, We1b, Wc2, bc2, Rc2, cb2, Wu2, bu2, We2a, We2b,
           Wc4, bc4, Rc4, cb4, Wf0, bf0, Wf1, bf1, We4a, We4b, Wfe0, bfe0,
           Wfe1, bfe1):
    B = z.shape[0]

    def relayout(Wc, din, dout):
        return Wc.reshape(EH, din, dout).transpose(1, 0, 2).reshape(din, EH * dout)

    wc1r = relayout(Wc1, 64, 64)
    wc2r = relayout(Wc2, 48, 64)
    wc4r = relayout(Wc4, 48, 64)
    row = lambda v: v.reshape(1, -1)
    ins = [z, Wi1, row(bi1), Wi2, row(bi2), row(Wlink), row(blink),
           We0a, We0b, wc1r, bc1.reshape(64, 64), Rc1, row(cb1),
           Wu1, row(bu1), We1a, We1b, wc2r, bc2.reshape(48, 64),
           Rc2, row(cb2), Wu2, row(bu2), We2a, We2b,
           wc4r, bc4.reshape(48, 64), Rc4, row(cb4), Wf0, row(bf0),
           Wf1, row(bf1), We4a, We4b, Wfe0, row(bfe0), Wfe1, row(bfe1)]

    in_specs = [pl.BlockSpec((BB, 128), lambda i: (i, 0))]
    in_specs += [_full(a.shape) for a in ins[1:]]

    node_out, edge_full = pl.pallas_call(
        _body,
        grid=(B // BB,),
        in_specs=in_specs,
        out_specs=[
            pl.BlockSpec((BB, 12, 16), lambda i: (i, 0, 0)),
            pl.BlockSpec((BB, 144, 4), lambda i: (i, 0, 0)),
        ],
        out_shape=[
            jax.ShapeDtypeStruct((B, 12, 16), jnp.float32),
            jax.ShapeDtypeStruct((B, 144, 4), jnp.float32),
        ],
    )(*ins)

    offdiag = np.array([i * 12 + j for i in range(12) for j in range(12)
                        if i != j], dtype=np.int32)
    return node_out, edge_full[:, offdiag, :]


# speedup vs baseline: 1.1436x; 1.0022x over previous
import jax
import jax.numpy as jnp
import numpy as np
from jax import lax
from jax.experimental import pallas as pl

BB = 16
EH = 64


def _leaky(x):
    return jnp.where(x >= 0, x, 0.05 * x)


def _pair_mask(n, rows):
    r = lax.broadcasted_iota(jnp.int32, (rows, 1), 0)
    p = r % (n * n)
    return jnp.where(p // n != p % n, 1.0, 0.0).astype(jnp.float32)


def _pairs(xf, n, d):
    bb = xf.shape[0]
    xs = jnp.broadcast_to(xf[:, :, None, :], (bb, n, n, d)).reshape(bb * n * n, d)
    xd = jnp.broadcast_to(xf[:, None, :, :], (bb, n, n, d)).reshape(bb * n * n, d)
    return xs, xd


def _edge_attr(xf, n, d, Wa, Wb):
    xs, xd = _pairs(xf, n, d)
    cat = jnp.concatenate([xs, xd], axis=-1)
    return cat, _leaky(_leaky(cat @ Wa) @ Wb)


def _conv(xf, n, din, dout, e_full, Wcr, bcm, R, cb):
    bb = xf.shape[0]
    x2 = xf.reshape(bb * n, din)
    Y = (x2 @ Wcr).reshape(bb * n, EH, dout)
    E3 = e_full.reshape(bb * n, n, EH)
    Z = lax.dot_general(E3, Y, (((2,), (1,)), ((0,), (0,))))
    agg = jnp.sum(Z.reshape(bb, n, n, dout), axis=1).reshape(bb * n, dout)
    xex = (jnp.sum(xf, axis=1, keepdims=True) - xf).reshape(bb * n, din)
    agg = (agg + xex @ bcm) * (1.0 / (n - 1))
    return _leaky(x2 @ R + cb + agg).reshape(bb, n, dout)


def _body(z_ref, wi1_ref, bi1_ref, wi2_ref, bi2_ref, wl_ref, blink_ref,
          we0a_ref, we0b_ref, wc1r_ref, bc1m_ref, rc1_ref, cb1_ref,
          wu1_ref, bu1_ref, we1a_ref, we1b_ref, wc2r_ref, bc2m_ref,
          rc2_ref, cb2_ref, wu2_ref, bu2_ref, we2a_ref, we2b_ref,
          wc4r_ref, bc4m_ref, rc4_ref, cb4_ref, wf0_ref, bf0_ref,
          wf1_ref, bf1_ref, we4a_ref, we4b_ref, wfe0_ref, bfe0_ref,
          wfe1_ref, bfe1_ref, node_ref, edge_ref):
    z = z_ref[...]
    h = _leaky(z @ wi1_ref[...] + bi1_ref[...])
    x0 = _leaky((h @ wi2_ref[...] + bi2_ref[...]).reshape(BB, 3, 64))

    cat0, e0 = _edge_attr(x0, 3, 64, we0a_ref[...], we0b_ref[...])
    gate = jax.nn.sigmoid(
        jnp.sum(cat0 * wl_ref[...], axis=-1, keepdims=True) + blink_ref[...])
    e0 = e0 * (gate * _pair_mask(3, BB * 9))
    x1 = _conv(x0, 3, 64, 64, e0, wc1r_ref[...], bc1m_ref[...],
               rc1_ref[...], cb1_ref[...])

    x1u = (x1.reshape(BB * 3, 64) @ wu1_ref[...] + bu1_ref[...])
    x1u = _leaky(x1u.reshape(BB * 3, 2, 48)).reshape(BB, 6, 48)
    _, e1 = _edge_attr(x1u, 6, 48, we1a_ref[...], we1b_ref[...])
    e1 = e1 * _pair_mask(6, BB * 36)
    x2 = _conv(x1u, 6, 48, 64, e1, wc2r_ref[...], bc2m_ref[...],
               rc2_ref[...], cb2_ref[...])

    x2u = (x2.reshape(BB * 6, 64) @ wu2_ref[...] + bu2_ref[...])
    x2u = _leaky(x2u.reshape(BB * 6, 2, 48)).reshape(BB, 12, 48)
    _, e2 = _edge_attr(x2u, 12, 48, we2a_ref[...], we2b_ref[...])
    e2 = e2 * _pair_mask(12, BB * 144)
    x3 = _conv(x2u, 12, 48, 64, e2, wc4r_ref[...], bc4m_ref[...],
               rc4_ref[...], cb4_ref[...])

    h0 = _leaky(x3.reshape(BB * 12, 64) @ wf0_ref[...] + bf0_ref[...])
    node = h0 @ wf1_ref[...] + bf1_ref[...]
    node_ref[...] = node.reshape(BB, 12, 16)

    _, e4 = _edge_attr(node.reshape(BB, 12, 16), 12, 16,
                       we4a_ref[...], we4b_ref[...])
    hs, hd = _pairs(h0.reshape(BB, 12, 64), 12, 64)
    pairf = 0.5 * (hs + hd)
    ef = _leaky(jnp.concatenate([e2, e4, pairf], axis=-1) @ wfe0_ref[...]
                + bfe0_ref[...])
    edge = ef @ wfe1_ref[...] + bfe1_ref[...]
    edge_ref[...] = edge.reshape(BB, 144, 4)


def _full(shape):
    nd = len(shape)
    return pl.BlockSpec(shape, lambda i: (0,) * nd)


def kernel(z, Wi1, bi1, Wi2, bi2, Wlink, blink, We0a, We0b, Wc1, bc1, Rc1, cb1,
           Wu1, bu1, We1a, We1b, Wc2, bc2, Rc2, cb2, Wu2, bu2, We2a, We2b,
           Wc4, bc4, Rc4, cb4, Wf0, bf0, Wf1, bf1, We4a, We4b, Wfe0, bfe0,
           Wfe1, bfe1):
    B = z.shape[0]

    def relayout(Wc, din, dout):
        return Wc.reshape(EH, din, dout).transpose(1, 0, 2).reshape(din, EH * dout)

    wc1r = relayout(Wc1, 64, 64)
    wc2r = relayout(Wc2, 48, 64)
    wc4r = relayout(Wc4, 48, 64)
    row = lambda v: v.reshape(1, -1)
    ins = [z, Wi1, row(bi1), Wi2, row(bi2), row(Wlink), row(blink),
           We0a, We0b, wc1r, bc1.reshape(64, 64), Rc1, row(cb1),
           Wu1, row(bu1), We1a, We1b, wc2r, bc2.reshape(48, 64),
           Rc2, row(cb2), Wu2, row(bu2), We2a, We2b,
           wc4r, bc4.reshape(48, 64), Rc4, row(cb4), Wf0, row(bf0),
           Wf1, row(bf1), We4a, We4b, Wfe0, row(bfe0), Wfe1, row(bfe1)]

    in_specs = [pl.BlockSpec((BB, 128), lambda i: (i, 0))]
    in_specs += [_full(a.shape) for a in ins[1:]]

    node_out, edge_full = pl.pallas_call(
        _body,
        grid=(B // BB,),
        in_specs=in_specs,
        out_specs=[
            pl.BlockSpec((BB, 12, 16), lambda i: (i, 0, 0)),
            pl.BlockSpec((BB, 144, 4), lambda i: (i, 0, 0)),
        ],
        out_shape=[
            jax.ShapeDtypeStruct((B, 12, 16), jnp.float32),
            jax.ShapeDtypeStruct((B, 144, 4), jnp.float32),
        ],
    )(*ins)

    offdiag = np.array([i * 12 + j for i in range(12) for j in range(12)
                        if i != j], dtype=np.int32)
    return node_out, edge_full[:, offdiag, :]
